# baseline (speedup 1.0000x reference)
import jax
import jax.numpy as jnp
from jax import lax
from jax.experimental import pallas as pl
from jax.experimental.pallas import tpu as pltpu
from jax.experimental.pallas import tpu_sc as plsc

_TIME_STEPS = 1000
_BATCH = 16384

_NC = 1
_NS = 16
_L = 16
_NW = _NC * _NS
_BPW = _BATCH // _NW


def _body(t_hbm, beta_hbm, alpha_hbm, beta_out, alpha_out,
          idx_v, beta_v, alpha_v, ob_v, oa_v, sem0, sem1, sem2):
    wid = lax.axis_index("s") * _NC + lax.axis_index("c")
    base = wid * _BPW
    c0 = pltpu.async_copy(t_hbm.at[pl.ds(base, _BPW)], idx_v, sem0)
    c1 = pltpu.async_copy(beta_hbm, beta_v, sem1)
    c2 = pltpu.async_copy(alpha_hbm, alpha_v, sem2)
    c0.wait()
    c1.wait()

    @plsc.parallel_loop(0, _BPW // _L, unroll=8)
    def _gather_beta(i):
        sl = pl.ds(i * _L, _L)
        ob_v[sl] = plsc.load_gather(beta_v, [idx_v[sl]])

    o1 = pltpu.async_copy(ob_v, beta_out.at[pl.ds(base, _BPW)], sem1)
    c2.wait()

    @plsc.parallel_loop(0, _BPW // _L, unroll=8)
    def _gather_alpha(i):
        sl = pl.ds(i * _L, _L)
        oa_v[sl] = plsc.load_gather(alpha_v, [idx_v[sl]])

    o2 = pltpu.async_copy(oa_v, alpha_out.at[pl.ds(base, _BPW)], sem2)
    o1.wait()
    o2.wait()


@jax.jit
def kernel(t, beta, alpha):
    f = pl.kernel(
        _body,
        out_type=(jax.ShapeDtypeStruct((_BATCH,), jnp.float32),
                  jax.ShapeDtypeStruct((_BATCH,), jnp.float32)),
        mesh=plsc.VectorSubcoreMesh(core_axis_name="c", subcore_axis_name="s",
                                    num_cores=_NC),
        compiler_params=pltpu.CompilerParams(
            needs_layout_passes=False, skip_device_barrier=True,
            use_tc_tiling_on_sc=False),
        scratch_types=[
            pltpu.VMEM((_BPW,), jnp.int32),
            pltpu.VMEM((_TIME_STEPS,), jnp.float32),
            pltpu.VMEM((_TIME_STEPS,), jnp.float32),
            pltpu.VMEM((_BPW,), jnp.float32),
            pltpu.VMEM((_BPW,), jnp.float32),
            pltpu.SemaphoreType.DMA,
            pltpu.SemaphoreType.DMA,
            pltpu.SemaphoreType.DMA,
        ],
    )
    return f(t, beta, alpha)


# speedup vs baseline: 1.0045x; 1.0045x over previous
import jax
import jax.numpy as jnp
from jax import lax
from jax.experimental import pallas as pl
from jax.experimental.pallas import tpu as pltpu
from jax.experimental.pallas import tpu_sc as plsc

_TIME_STEPS = 1000
_BATCH = 16384

_NC = 1
_NS = 16
_L = 16
_NW = _NC * _NS
_BPW = _BATCH // _NW


def _body(t_hbm, beta_hbm, alpha_hbm, beta_out, alpha_out,
          idx_v, beta_v, alpha_v, ob_v, oa_v, sem0, sem1, sem2):
    wid = lax.axis_index("s") * _NC + lax.axis_index("c")
    base = wid * _BPW
    c0 = pltpu.async_copy(t_hbm.at[pl.ds(base, _BPW)], idx_v, sem0)
    c1 = pltpu.async_copy(beta_hbm, beta_v, sem1)
    c2 = pltpu.async_copy(alpha_hbm, alpha_v, sem2)
    c0.wait()
    c1.wait()

    @plsc.parallel_loop(0, _BPW // _L, unroll=8)
    def _gather_beta(i):
        sl = pl.ds(i * _L, _L)
        ob_v[sl] = plsc.load_gather(beta_v, [idx_v[sl]])

    o1 = pltpu.async_copy(ob_v, beta_out.at[pl.ds(base, _BPW)], sem1)
    c2.wait()

    @plsc.parallel_loop(0, _BPW // _L, unroll=8)
    def _gather_alpha(i):
        sl = pl.ds(i * _L, _L)
        oa_v[sl] = plsc.load_gather(alpha_v, [idx_v[sl]])

    o2 = pltpu.async_copy(oa_v, alpha_out.at[pl.ds(base, _BPW)], sem2)
    o1.wait()
    o2.wait()


@jax.jit
def kernel(t, beta, alpha):
    f = pl.kernel(
        _body,
        out_type=(jax.ShapeDtypeStruct((_BATCH,), jnp.float32),
                  jax.ShapeDtypeStruct((_BATCH,), jnp.float32)),
        mesh=plsc.VectorSubcoreMesh(core_axis_name="c", subcore_axis_name="s",
                                    num_cores=_NC),
        compiler_params=pltpu.CompilerParams(needs_layout_passes=False),
        scratch_types=[
            pltpu.VMEM((_BPW,), jnp.int32),
            pltpu.VMEM((_TIME_STEPS,), jnp.float32),
            pltpu.VMEM((_TIME_STEPS,), jnp.float32),
            pltpu.VMEM((_BPW,), jnp.float32),
            pltpu.VMEM((_BPW,), jnp.float32),
            pltpu.SemaphoreType.DMA,
            pltpu.SemaphoreType.DMA,
            pltpu.SemaphoreType.DMA,
        ],
    )
    return f(t, beta, alpha)
